# Initial kernel scaffold; baseline (speedup 1.0000x reference)
import functools

import jax
import jax.numpy as jnp
from jax import lax
from jax.experimental import pallas as pl
from jax.experimental.pallas import tpu as pltpu
from jax.experimental.pallas import tpu_sc as plsc

_VOCAB = 1000000
_EMBED = 128
_HIDDEN = 128
_OUTPUTS = 91
_BATCH = 16384
_CTX = 6
_N_ROWS = _BATCH * _CTX

_INFO = plsc.get_sparse_core_info()
_NC = _INFO.num_cores
_NS = _INFO.num_subcores
_NW = _NC * _NS

_B_PER_W = _N_ROWS // _NW
_CH = 512
_N_CH = _B_PER_W // _CH

_sc_mesh = plsc.VectorSubcoreMesh(core_axis_name="c", subcore_axis_name="s")


@functools.partial(
    pl.kernel,
    mesh=_sc_mesh,
    out_type=jax.ShapeDtypeStruct((_N_ROWS, _EMBED), jnp.float32),
    scratch_types=[
        pltpu.VMEM((_B_PER_W,), jnp.int32),
        pltpu.VMEM((_CH, _EMBED), jnp.float32),
        pltpu.SemaphoreType.DMA,
    ],
)
def _sc_gather(idx_hbm, table_hbm, out_hbm, idx_v, rows_v, sem):
    wid = lax.axis_index("s") * _NC + lax.axis_index("c")
    base = wid * _B_PER_W
    pltpu.sync_copy(idx_hbm.at[pl.ds(base, _B_PER_W)], idx_v)
    for c in range(_N_CH):
        pltpu.async_copy(
            table_hbm.at[idx_v.at[pl.ds(c * _CH, _CH)]], rows_v, sem
        ).wait()
        pltpu.sync_copy(rows_v, out_hbm.at[pl.ds(base + c * _CH, _CH)])


def _mlp_body(x_ref, w1_ref, b1_ref, w2_ref, b2_ref, o_ref):
    h = jnp.dot(x_ref[...], w1_ref[...], preferred_element_type=jnp.float32)
    h = jnp.maximum(h + b1_ref[...], 0.0)
    o_ref[...] = (
        jnp.dot(h, w2_ref[...], preferred_element_type=jnp.float32) + b2_ref[...]
    )


_BM = 1024


def _mlp(x, W1, b1, W2, b2):
    din = _CTX * _EMBED
    return pl.pallas_call(
        _mlp_body,
        grid=(_BATCH // _BM,),
        in_specs=[
            pl.BlockSpec((_BM, din), lambda i: (i, 0)),
            pl.BlockSpec((din, _HIDDEN), lambda i: (0, 0)),
            pl.BlockSpec((1, _HIDDEN), lambda i: (0, 0)),
            pl.BlockSpec((_HIDDEN, _OUTPUTS), lambda i: (0, 0)),
            pl.BlockSpec((1, _OUTPUTS), lambda i: (0, 0)),
        ],
        out_specs=pl.BlockSpec((_BM, _OUTPUTS), lambda i: (i, 0)),
        out_shape=jax.ShapeDtypeStruct((_BATCH, _OUTPUTS), jnp.float32),
    )(x, W1, b1.reshape(1, -1), W2, b2.reshape(1, -1))


def kernel(inputs, table, W1, b1, W2, b2):
    idx = inputs.reshape(-1)
    embeds = _sc_gather(idx, table)
    x = embeds.reshape(_BATCH, _CTX * _EMBED)
    return _mlp(x, W1, b1, W2, b2)


# speedup vs baseline: 11.9386x; 11.9386x over previous
import functools

import jax
import jax.numpy as jnp
from jax import lax
from jax.experimental import pallas as pl
from jax.experimental.pallas import tpu as pltpu
from jax.experimental.pallas import tpu_sc as plsc

_VOCAB = 1000000
_EMBED = 128
_HIDDEN = 128
_OUTPUTS = 91
_BATCH = 16384
_CTX = 6
_N_ROWS = _BATCH * _CTX

_INFO = plsc.get_sparse_core_info()
_NC = _INFO.num_cores
_NS = _INFO.num_subcores
_NW = _NC * _NS

_B_PER_W = _N_ROWS // _NW
_CH = 384
_N_CH = _B_PER_W // _CH

_sc_mesh = plsc.VectorSubcoreMesh(core_axis_name="c", subcore_axis_name="s")


@functools.partial(
    pl.kernel,
    mesh=_sc_mesh,
    out_type=jax.ShapeDtypeStruct((_N_ROWS, _EMBED), jnp.float32),
    scratch_types=[
        pltpu.VMEM((_B_PER_W,), jnp.int32),
        pltpu.VMEM((2, _CH, _EMBED), jnp.float32),
        pltpu.SemaphoreType.DMA,
        pltpu.SemaphoreType.DMA,
    ],
)
def _sc_gather(idx_hbm, table_hbm, out_hbm, idx_v, rows_v, sem0, sem1):
    wid = lax.axis_index("s") * _NC + lax.axis_index("c")
    base = wid * _B_PER_W
    pltpu.sync_copy(idx_hbm.at[pl.ds(base, _B_PER_W)], idx_v)
    sems = (sem0, sem1)
    pending = pltpu.async_copy(
        table_hbm.at[idx_v.at[pl.ds(0, _CH)]], rows_v.at[0], sems[0]
    )
    for c in range(_N_CH):
        nxt = None
        if c + 1 < _N_CH:
            nxt = pltpu.async_copy(
                table_hbm.at[idx_v.at[pl.ds((c + 1) * _CH, _CH)]],
                rows_v.at[(c + 1) % 2],
                sems[(c + 1) % 2],
            )
        pending.wait()
        pltpu.sync_copy(rows_v.at[c % 2], out_hbm.at[pl.ds(base + c * _CH, _CH)])
        pending = nxt


def _mlp_body(x_ref, w1_ref, b1_ref, w2_ref, b2_ref, o_ref):
    h = jnp.dot(x_ref[...], w1_ref[...], preferred_element_type=jnp.float32)
    h = jnp.maximum(h + b1_ref[...], 0.0)
    o_ref[...] = (
        jnp.dot(h, w2_ref[...], preferred_element_type=jnp.float32) + b2_ref[...]
    )


_BM = 1024


def _mlp(x, W1, b1, W2, b2):
    din = _CTX * _EMBED
    return pl.pallas_call(
        _mlp_body,
        grid=(_BATCH // _BM,),
        in_specs=[
            pl.BlockSpec((_BM, din), lambda i: (i, 0)),
            pl.BlockSpec((din, _HIDDEN), lambda i: (0, 0)),
            pl.BlockSpec((1, _HIDDEN), lambda i: (0, 0)),
            pl.BlockSpec((_HIDDEN, _OUTPUTS), lambda i: (0, 0)),
            pl.BlockSpec((1, _OUTPUTS), lambda i: (0, 0)),
        ],
        out_specs=pl.BlockSpec((_BM, _OUTPUTS), lambda i: (i, 0)),
        out_shape=jax.ShapeDtypeStruct((_BATCH, _OUTPUTS), jnp.float32),
    )(x, W1, b1.reshape(1, -1), W2, b2.reshape(1, -1))


def kernel(inputs, table, W1, b1, W2, b2):
    idx = inputs.reshape(-1)
    embeds = _sc_gather(idx, table)
    x = embeds.reshape(_BATCH, _CTX * _EMBED)
    return _mlp(x, W1, b1, W2, b2)


# speedup vs baseline: 20.3068x; 1.7009x over previous
import functools

import jax
import jax.numpy as jnp
from jax import lax
from jax.experimental import pallas as pl
from jax.experimental.pallas import tpu as pltpu
from jax.experimental.pallas import tpu_sc as plsc

_VOCAB = 1000000
_EMBED = 128
_HIDDEN = 128
_OUTPUTS = 91
_BATCH = 16384
_CTX = 6
_N_ROWS = _BATCH * _CTX

_INFO = plsc.get_sparse_core_info()
_NC = _INFO.num_cores
_NS = _INFO.num_subcores
_NW = _NC * _NS

_B_PER_W = _N_ROWS // _NW
_CH = 384
_N_CH = _B_PER_W // _CH

_sc_mesh = plsc.VectorSubcoreMesh(core_axis_name="c", subcore_axis_name="s")


@functools.partial(
    pl.kernel,
    mesh=_sc_mesh,
    out_type=jax.ShapeDtypeStruct((_N_ROWS, _EMBED), jnp.float32),
    scratch_types=[
        pltpu.VMEM((_B_PER_W,), jnp.int32),
        pltpu.VMEM((2, _CH, _EMBED), jnp.float32),
        pltpu.SemaphoreType.DMA,
        pltpu.SemaphoreType.DMA,
    ],
)
def _sc_gather(idx_hbm, table_hbm, out_hbm, idx_v, rows_v, sem0, sem1):
    wid = lax.axis_index("s") * _NC + lax.axis_index("c")
    base = wid * _B_PER_W
    pltpu.sync_copy(idx_hbm.at[pl.ds(base, _B_PER_W)], idx_v)
    sems = (sem0, sem1)
    pending = pltpu.async_copy(
        table_hbm.at[idx_v.at[pl.ds(0, _CH)]], rows_v.at[0], sems[0]
    )
    for c in range(_N_CH):
        nxt = None
        if c + 1 < _N_CH:
            nxt = pltpu.async_copy(
                table_hbm.at[idx_v.at[pl.ds((c + 1) * _CH, _CH)]],
                rows_v.at[(c + 1) % 2],
                sems[(c + 1) % 2],
            )
        pending.wait()
        pltpu.sync_copy(rows_v.at[c % 2], out_hbm.at[pl.ds(base + c * _CH, _CH)])
        pending = nxt


def _mlp_body(x_ref, w1_ref, b1_ref, w2_ref, b2_ref, o_ref):
    h = jnp.dot(x_ref[0], w1_ref[0], preferred_element_type=jnp.float32)
    for j in range(1, _CTX):
        h = h + jnp.dot(x_ref[j], w1_ref[j], preferred_element_type=jnp.float32)
    h = jnp.maximum(h + b1_ref[...], 0.0)
    o_ref[...] = (
        jnp.dot(h, w2_ref[...], preferred_element_type=jnp.float32) + b2_ref[...]
    )


_BM = 1024


def _mlp(x, W1, b1, W2, b2):
    return pl.pallas_call(
        _mlp_body,
        grid=(_BATCH // _BM,),
        in_specs=[
            pl.BlockSpec((_CTX, _BM, _EMBED), lambda i: (0, i, 0)),
            pl.BlockSpec((_CTX, _EMBED, _HIDDEN), lambda i: (0, 0, 0)),
            pl.BlockSpec((1, _HIDDEN), lambda i: (0, 0)),
            pl.BlockSpec((_HIDDEN, _OUTPUTS), lambda i: (0, 0)),
            pl.BlockSpec((1, _OUTPUTS), lambda i: (0, 0)),
        ],
        out_specs=pl.BlockSpec((_BM, _OUTPUTS), lambda i: (i, 0)),
        out_shape=jax.ShapeDtypeStruct((_BATCH, _OUTPUTS), jnp.float32),
    )(x, W1, b1.reshape(1, -1), W2, b2.reshape(1, -1))


def kernel(inputs, table, W1, b1, W2, b2):
    idx_t = inputs.T.reshape(-1)
    embeds = _sc_gather(idx_t, table)
    x = embeds.reshape(_CTX, _BATCH, _EMBED)
    w1 = W1.reshape(_CTX, _EMBED, _HIDDEN)
    return _mlp(x, w1, b1, W2, b2)
